# baseline (speedup 1.0000x reference)
import jax
import jax.numpy as jnp
from jax.experimental import pallas as pl

N = 10000
E = 320000
DF = 128
DE = 16
HID = 128
HEADS = 4
OUT = 128
G = 64


def _bn(z, g, b):
    return z * (g / jnp.sqrt(1.0 + 1e-5)) + b


def _fc_kernel(hh_ref, w1_ref, b1_ref, w2_ref, b2_ref, o_ref):
    h = jnp.maximum(jnp.dot(hh_ref[...], w1_ref[...],
                            preferred_element_type=jnp.float32) + b1_ref[...], 0.0)
    o_ref[...] = jnp.dot(h, w2_ref[...],
                         preferred_element_type=jnp.float32) + b2_ref[...]


def kernel(x, edge_index, edge_attr, batch, params):
    p = params
    src = edge_index[0]
    dst = edge_index[1]
    edge_emb = jax.nn.relu(edge_attr @ p["We1"] + p["be1"]) @ p["We2"] + p["be2"]
    h = x @ p["Win1"] + p["bin1"]
    h = jax.nn.relu(_bn(h, p["gin"], p["bbn_in"]))
    h = jax.nn.relu(h @ p["Win2"] + p["bin2"])
    loop = jnp.arange(N, dtype=src.dtype)
    s2 = jnp.concatenate([src, loop])
    d2 = jnp.concatenate([dst, loop])
    xh = (h @ p["Wgat"]).reshape(N, HEADS, HID)
    a_src = jnp.sum(xh * p["att_src"], axis=-1)
    a_dst = jnp.sum(xh * p["att_dst"], axis=-1)
    alpha = jax.nn.leaky_relu(a_src[s2] + a_dst[d2], 0.2)
    amax = jax.ops.segment_max(alpha, d2, num_segments=N)
    amax = jnp.where(jnp.isfinite(amax), amax, 0.0)
    ae = jnp.exp(alpha - amax[d2])
    denom = jax.ops.segment_sum(ae, d2, num_segments=N)
    att = ae / (denom[d2] + 1e-16)
    h1 = jax.ops.segment_sum(xh[s2] * att[:, :, None], d2, num_segments=N).mean(axis=1) + p["bgat"]

    def hidden_nn(z):
        z = z @ p["Wh1"] + p["bh1"]
        z = jax.nn.relu(_bn(z, p["gh"], p["bbn_h"]))
        return jax.nn.relu(z @ p["Wh2"] + p["bh2"])

    def gine(hn, Wl, bl):
        msg = jax.nn.relu(hn[src] + (edge_emb @ Wl + bl))
        agg = jax.ops.segment_sum(msg, dst, num_segments=N)
        return hidden_nn(hn + agg)

    h2 = gine(h1, p["Wgl2"], p["bgl2"])
    h3 = gine(h2, p["Wgl3"], p["bgl3"])
    p1 = jax.ops.segment_sum(h1, batch, num_segments=G)
    p2 = jax.ops.segment_sum(h2, batch, num_segments=G)
    p3 = jax.ops.segment_sum(h3, batch, num_segments=G)
    hh = jnp.concatenate([p1, p2, p3], axis=1)

    out = pl.pallas_call(
        _fc_kernel,
        out_shape=jax.ShapeDtypeStruct((G, OUT), jnp.float32),
    )(hh, p["Wfc1"], p["bfc1"], p["Wfc2"], p["bfc2"])
    return out


# speedup vs baseline: 15.1130x; 15.1130x over previous
import dataclasses
import functools

import jax
import jax.numpy as jnp
from jax import lax
from jax.experimental import pallas as pl
from jax.experimental.pallas import tpu as pltpu
from jax.experimental.pallas import tpu_sc as plsc

N = 10000
E = 320000
DF = 128
DE = 16
HID = 128
HEADS = 4
OUT = 128
G = 64

NC = 2
NS = 16
L = 16
NW = NC * NS
EPW = E // NW
BE = 128
NFULL = EPW // BE
TAIL = EPW - NFULL * BE
ROWS_PER_SUB = 624
ROWS_TAIL = N - NS * ROWS_PER_SUB

B2 = 64
NF2 = EPW // B2
TL2 = EPW - NF2 * B2

_MESH = dict(core_axis_name="c", subcore_axis_name="s")


def _sc_params_no_layout():
    cp = pltpu.CompilerParams()
    if "needs_layout_passes" in pltpu.CompilerParams.__dataclass_fields__:
        cp = dataclasses.replace(cp, needs_layout_passes=False)
    return cp


def _bn(z, g, b):
    return z * (g / jnp.sqrt(1.0 + 1e-5)) + b


def _dot(a, b):
    return jax.lax.dot_general(a, b, (((1,), (0,)), ((), ())),
                               preferred_element_type=jnp.float32)


def _edge_body(ea_ref, we1, be1, we2, be2, wl2, bl2, wl3, bl3,
               el2_ref, el3_ref):
    t = jnp.maximum(_dot(ea_ref[...], we1[...]) + be1[...], 0.0)
    emb = _dot(t, we2[...]) + be2[...]
    el2_ref[...] = _dot(emb, wl2[...]) + bl2[...]
    el3_ref[...] = _dot(emb, wl3[...]) + bl3[...]


def _node_body(x_ref, win1, bin1, gin, bbn_in, win2, bin2, wgat,
               asrc_w, adst_w,
               h_ref, xh_ref, asrc_ref, adst_ref, gmax_ref):
    h = _dot(x_ref[...], win1[...]) + bin1[...]
    h = jnp.maximum(_bn(h, gin[...], bbn_in[...]), 0.0)
    h = jnp.maximum(_dot(h, win2[...]) + bin2[...], 0.0)
    h_ref[...] = h
    xh = _dot(h, wgat[...])
    xh_ref[...] = xh
    a_src = _dot(xh, asrc_w[...])
    a_dst = _dot(xh, adst_w[...])
    asrc_ref[...] = a_src
    adst_ref[...] = a_dst
    gmax = jnp.max(a_src, axis=0, keepdims=True)
    gmax16 = jnp.concatenate(
        [gmax, jnp.full((1, 16 - HEADS), -jnp.inf, jnp.float32)], axis=1)

    @pl.when(pl.program_id(0) == 0)
    def _():
        gmax_ref[...] = gmax16

    @pl.when(pl.program_id(0) != 0)
    def _():
        gmax_ref[...] = jnp.maximum(gmax_ref[...], gmax16)


def _invden_body(asrc_ref, adst_ref, gmax_ref, dsum_ref,
                 aeloop_ref, invden_ref):
    a_src = asrc_ref[...]
    a_dst = adst_ref[...]
    gmax = gmax_ref[...][:, :HEADS]
    z = a_src + a_dst
    alpha = jnp.maximum(z, 0.2 * z)
    zs = a_dst + gmax
    shift = jnp.maximum(zs, 0.2 * zs)
    ae_loop = jnp.exp(alpha - shift)
    aeloop_ref[...] = ae_loop
    den = dsum_ref[...] + ae_loop
    invden_ref[...] = 1.0 / (den + 1e-16)


def _h1_body(p0_ref, p1_ref, aeloop_ref, invden_ref, xh_ref, bgat, h1_ref):
    w = aeloop_ref[...] * invden_ref[...]
    xh = xh_ref[...]
    selfc = w[:, 0:1] * xh[:, :HID]
    for kk in range(1, HEADS):
        selfc = selfc + w[:, kk:kk + 1] * xh[:, kk * HID:(kk + 1) * HID]
    h1_ref[...] = 0.25 * (p0_ref[...] + p1_ref[...] + selfc) + bgat[...]


def _hidden_body(hn_ref, p0_ref, p1_ref, wh1, bh1, gh, bbn_h, wh2, bh2,
                 out_ref):
    z = hn_ref[...] + p0_ref[...] + p1_ref[...]
    z = _dot(z, wh1[...]) + bh1[...]
    z = jnp.maximum(_bn(z, gh[...], bbn_h[...]), 0.0)
    out_ref[...] = jnp.maximum(_dot(z, wh2[...]) + bh2[...], 0.0)


def _pool_fc_body(h1_ref, h2_ref, h3_ref, b_ref, w1, b1, w2, b2, o_ref):
    biota = jax.lax.broadcasted_iota(jnp.int32, (G, 1), 0)
    oh = (biota == b_ref[...]).astype(jnp.float32)
    p1 = _dot(oh, h1_ref[...])
    p2 = _dot(oh, h2_ref[...])
    p3 = _dot(oh, h3_ref[...])
    hh = jnp.concatenate([p1, p2, p3], axis=1)
    hh = jnp.maximum(_dot(hh, w1[...]) + b1[...], 0.0)
    o_ref[...] = _dot(hh, w2[...]) + b2[...]


def _wid(core, sub):
    return sub * NC + core


def _zero_spmem(zsrc, acc_sh, sub, blk):
    for j in range((ROWS_PER_SUB + blk - 1) // blk):
        sz = min(blk, ROWS_PER_SUB - j * blk)
        pltpu.sync_copy(zsrc.at[pl.ds(0, sz)],
                        acc_sh.at[pl.ds(sub * ROWS_PER_SUB + j * blk, sz)])

    @pl.when(sub == NS - 1)
    def _():
        pltpu.sync_copy(zsrc.at[pl.ds(0, ROWS_TAIL)],
                        acc_sh.at[pl.ds(NS * ROWS_PER_SUB, ROWS_TAIL)])


def _write_out_spmem(acc_sh, out_hbm, core, sub):
    pltpu.sync_copy(acc_sh.at[pl.ds(sub * ROWS_PER_SUB, ROWS_PER_SUB)],
                    out_hbm.at[core, pl.ds(sub * ROWS_PER_SUB, ROWS_PER_SUB)])

    @pl.when(sub == NS - 1)
    def _():
        pltpu.sync_copy(acc_sh.at[pl.ds(NS * ROWS_PER_SUB, ROWS_TAIL)],
                        out_hbm.at[core, pl.ds(NS * ROWS_PER_SUB, ROWS_TAIL)])


def _gat_p1a_sc(asrc, adst, gmax_tab, src, dst):
    @functools.partial(
        pl.kernel,
        out_type=(jax.ShapeDtypeStruct((E * 16,), jnp.float32),
                  jax.ShapeDtypeStruct((NW, N * HEADS), jnp.float32)),
        mesh=plsc.VectorSubcoreMesh(**_MESH),
        scratch_types=[
            pltpu.VMEM((N * HEADS,), jnp.float32),
            pltpu.VMEM((N * HEADS,), jnp.float32),
            pltpu.VMEM((N * HEADS,), jnp.float32),
            pltpu.VMEM((1, 16), jnp.float32),
            pltpu.VMEM((BE,), jnp.int32),
            pltpu.VMEM((BE,), jnp.int32),
            pltpu.VMEM((TAIL,), jnp.int32),
            pltpu.VMEM((TAIL,), jnp.int32),
            pltpu.VMEM((BE * 16,), jnp.float32),
        ],
        compiler_params=_sc_params_no_layout(),
    )
    def k(asrc_hbm, adst_hbm, gmax_hbm, src_hbm, dst_hbm, ae_hbm, den_hbm,
          asrc_v, adst_v, denloc, gmax_v, sidx, didx, sidx_t, didx_t, aeblk):
        core = lax.axis_index("c")
        sub = lax.axis_index("s")
        wid = _wid(core, sub)
        pltpu.sync_copy(asrc_hbm, asrc_v)
        pltpu.sync_copy(adst_hbm, adst_v)
        pltpu.sync_copy(gmax_hbm, gmax_v)

        @pl.loop(0, BE * 16 // L)
        def _(i):
            aeblk.at[pl.ds(i * L, L)][...] = jnp.zeros((L,), jnp.float32)

        @pl.loop(0, N * HEADS // L)
        def _(i):
            denloc.at[pl.ds(i * L, L)][...] = jnp.zeros((L,), jnp.float32)

        z16 = jnp.zeros((L,), jnp.int32)
        gm = [plsc.load_gather(gmax_v, [z16, jnp.full((L,), kk, jnp.int32)])
              for kk in range(HEADS)]

        def do_chunk(base, nb, sidx_r, didx_r):
            pltpu.sync_copy(src_hbm.at[pl.ds(base, nb)], sidx_r)
            pltpu.sync_copy(dst_hbm.at[pl.ds(base, nb)], didx_r)
            for g in range(nb // L):
                sv = sidx_r.at[pl.ds(g * L, L)][...] * HEADS
                dv = didx_r.at[pl.ds(g * L, L)][...] * HEADS
                rid16 = (lax.iota(jnp.int32, L) + g * L) * 16
                for kk in range(HEADS):
                    as_v = plsc.load_gather(asrc_v, [sv + kk])
                    ad_v = plsc.load_gather(adst_v, [dv + kk])
                    z = as_v + ad_v
                    alpha = jnp.maximum(z, 0.2 * z)
                    zs = ad_v + gm[kk]
                    sh = jnp.maximum(zs, 0.2 * zs)
                    ae_v = jnp.exp(alpha - sh)
                    plsc.store_scatter(aeblk, [rid16 + kk], ae_v)
                    plsc.addupdate_scatter(denloc, [dv + kk], ae_v)
            aeblk_r = aeblk if nb == BE else aeblk.at[pl.ds(0, nb * 16)]
            pltpu.sync_copy(aeblk_r, ae_hbm.at[pl.ds(base * 16, nb * 16)])

        @pl.loop(0, NFULL)
        def _(j):
            do_chunk(wid * EPW + j * BE, BE, sidx, didx)

        if TAIL:
            do_chunk(wid * EPW + NFULL * BE, TAIL, sidx_t, didx_t)

        pltpu.sync_copy(denloc, den_hbm.at[wid])

    return k(asrc, adst, gmax_tab, src, dst)


def _gat_p2a_sc(ae, invden, dst):
    @functools.partial(
        pl.kernel,
        out_type=jax.ShapeDtypeStruct((E * HEADS * L,), jnp.float32),
        mesh=plsc.VectorSubcoreMesh(**_MESH),
        scratch_types=[
            pltpu.VMEM((N * HEADS,), jnp.float32),
            pltpu.VMEM((BE,), jnp.int32),
            pltpu.VMEM((TAIL,), jnp.int32),
            pltpu.VMEM((BE * 16,), jnp.float32),
            pltpu.VMEM((BE * HEADS * L,), jnp.float32),
        ],
        compiler_params=_sc_params_no_layout(),
    )
    def k(ae_hbm, inv_hbm, dst_hbm, att_hbm, inv_v, didx, didx_t, aeb, attb):
        core = lax.axis_index("c")
        sub = lax.axis_index("s")
        wid = _wid(core, sub)
        pltpu.sync_copy(inv_hbm, inv_v)

        def do_chunk(base, nb, didx_r):
            aeb_r = aeb if nb == BE else aeb.at[pl.ds(0, nb * 16)]
            attb_r = attb if nb == BE else attb.at[pl.ds(0, nb * HEADS * L)]
            pltpu.sync_copy(dst_hbm.at[pl.ds(base, nb)], didx_r)
            pltpu.sync_copy(ae_hbm.at[pl.ds(base * 16, nb * 16)], aeb_r)
            for g in range(nb // L):
                dv = didx_r.at[pl.ds(g * L, L)][...] * HEADS
                rid16 = (lax.iota(jnp.int32, L) + g * L) * 16
                rid64 = (lax.iota(jnp.int32, L) + g * L) * (HEADS * L)
                for kk in range(HEADS):
                    aev = plsc.load_gather(aeb, [rid16 + kk])
                    ivv = plsc.load_gather(inv_v, [dv + kk])
                    att = aev * ivv
                    for t in range(L):
                        plsc.store_scatter(attb, [rid64 + (kk * L + t)], att)
            pltpu.sync_copy(attb_r,
                            att_hbm.at[pl.ds(base * HEADS * L,
                                             nb * HEADS * L)])

        @pl.loop(0, NFULL)
        def _(j):
            do_chunk(wid * EPW + j * BE, BE, didx)

        if TAIL:
            do_chunk(wid * EPW + NFULL * BE, TAIL, didx_t)

    return k(ae, invden, dst)


def _gat_p2b_sc(xh2d, att16, src, dst):
    @functools.partial(
        pl.kernel,
        out_type=jax.ShapeDtypeStruct((NC, N, HID), jnp.float32),
        mesh=plsc.VectorSubcoreMesh(**_MESH),
        scratch_types=[
            pltpu.VMEM_SHARED((N, HID), jnp.float32),
            pltpu.VMEM((B2,), jnp.int32),
            pltpu.VMEM((B2,), jnp.int32),
            pltpu.VMEM((TL2,), jnp.int32),
            pltpu.VMEM((TL2,), jnp.int32),
            pltpu.VMEM((B2 * HEADS * L,), jnp.float32),
            pltpu.VMEM((B2, HEADS * HID), jnp.float32),
            pltpu.VMEM((B2, HID), jnp.float32),
            pltpu.SemaphoreType.DMA,
        ],
    )
    def k(xh_hbm, att_hbm, src_hbm, dst_hbm, out_hbm,
          acc_sh, sidx, didx, sidx_t, didx_t, attb, rows, msg, sem):
        core = lax.axis_index("c")
        sub = lax.axis_index("s")
        wid = _wid(core, sub)

        @pl.loop(0, B2)
        def _(i):
            for c in range(HID // L):
                msg.at[i, pl.ds(c * L, L)][...] = jnp.zeros((L,), jnp.float32)

        _zero_spmem(msg, acc_sh, sub, B2)
        plsc.subcore_barrier()

        def do_chunk(base, nb, sidx_r, didx_r):
            rows_r = rows if nb == B2 else rows.at[pl.ds(0, nb)]
            attb_r = attb if nb == B2 else attb.at[pl.ds(0, nb * HEADS * L)]
            msg_r = msg if nb == B2 else msg.at[pl.ds(0, nb)]
            pltpu.sync_copy(src_hbm.at[pl.ds(base, nb)], sidx_r)
            pltpu.sync_copy(dst_hbm.at[pl.ds(base, nb)], didx_r)
            pltpu.sync_copy(att_hbm.at[pl.ds(base * HEADS * L,
                                             nb * HEADS * L)], attb_r)
            pltpu.async_copy(xh_hbm.at[sidx_r], rows_r, sem).wait()

            @pl.loop(0, nb)
            def _(e):
                av = [attb.at[pl.ds(e * (HEADS * L) + kk * L, L)][...]
                      for kk in range(HEADS)]
                for c in range(HID // L):
                    acc = av[0] * rows.at[e, pl.ds(c * L, L)][...]
                    for kk in range(1, HEADS):
                        acc = acc + av[kk] * rows.at[
                            e, pl.ds(kk * HID + c * L, L)][...]
                    msg.at[e, pl.ds(c * L, L)][...] = acc

            pltpu.sync_copy(msg_r, acc_sh.at[didx_r], add=True)

        @pl.loop(0, NF2)
        def _(j):
            do_chunk(wid * EPW + j * B2, B2, sidx, didx)

        if TL2:
            do_chunk(wid * EPW + NF2 * B2, TL2, sidx_t, didx_t)

        plsc.subcore_barrier()
        _write_out_spmem(acc_sh, out_hbm, core, sub)

    return k(xh2d, att16, src, dst)


def _gine_agg_sc(hn, el, src, dst):
    @functools.partial(
        pl.kernel,
        out_type=jax.ShapeDtypeStruct((NC, N, HID), jnp.float32),
        mesh=plsc.VectorSubcoreMesh(**_MESH),
        scratch_types=[
            pltpu.VMEM_SHARED((N, HID), jnp.float32),
            pltpu.VMEM((BE,), jnp.int32),
            pltpu.VMEM((BE,), jnp.int32),
            pltpu.VMEM((TAIL,), jnp.int32),
            pltpu.VMEM((TAIL,), jnp.int32),
            pltpu.VMEM((BE, HID), jnp.float32),
            pltpu.VMEM((BE, HID), jnp.float32),
            pltpu.SemaphoreType.DMA,
        ],
    )
    def k(hn_hbm, el_hbm, src_hbm, dst_hbm, out_hbm, acc_sh, sidx, didx,
          sidx_t, didx_t, rows, elb, sem):
        core = lax.axis_index("c")
        sub = lax.axis_index("s")
        wid = _wid(core, sub)

        @pl.loop(0, BE)
        def _(i):
            for c in range(HID // L):
                rows.at[i, pl.ds(c * L, L)][...] = jnp.zeros((L,), jnp.float32)

        _zero_spmem(rows, acc_sh, sub, BE)
        plsc.subcore_barrier()

        def do_chunk(base, nb, sidx_r, didx_r):
            rows_r = rows if nb == BE else rows.at[pl.ds(0, nb)]
            elb_r = elb if nb == BE else elb.at[pl.ds(0, nb)]
            pltpu.sync_copy(src_hbm.at[pl.ds(base, nb)], sidx_r)
            pltpu.sync_copy(dst_hbm.at[pl.ds(base, nb)], didx_r)
            pltpu.sync_copy(el_hbm.at[pl.ds(base, nb)], elb_r)
            pltpu.async_copy(hn_hbm.at[sidx_r], rows_r, sem).wait()

            @pl.loop(0, nb)
            def _(i):
                for c in range(HID // L):
                    slc = (i, pl.ds(c * L, L))
                    rows.at[slc][...] = jnp.maximum(
                        rows.at[slc][...] + elb.at[slc][...], 0.0)

            pltpu.sync_copy(rows_r, acc_sh.at[didx_r], add=True)

        @pl.loop(0, NFULL)
        def _(j):
            do_chunk(wid * EPW + j * BE, BE, sidx, didx)

        if TAIL:
            do_chunk(wid * EPW + NFULL * BE, TAIL, sidx_t, didx_t)

        plsc.subcore_barrier()
        _write_out_spmem(acc_sh, out_hbm, core, sub)

    return k(hn, el, src, dst)


EBLK = 8000


def _full_spec(shape):
    nd = len(shape)
    return pl.BlockSpec(shape, lambda i, _nd=nd: (0,) * _nd)


def kernel(x, edge_index, edge_attr, batch, params):
    p = params
    src = edge_index[0]
    dst = edge_index[1]
    eye = jnp.eye(HEADS, dtype=jnp.float32)
    asrc_w = (p["att_src"][:, :, None] * eye[:, None, :]).reshape(HEADS * HID, HEADS)
    adst_w = (p["att_dst"][:, :, None] * eye[:, None, :]).reshape(HEADS * HID, HEADS)

    ew = (p["We1"], p["be1"], p["We2"], p["be2"],
          p["Wgl2"], p["bgl2"], p["Wgl3"], p["bgl3"])
    el2, el3 = pl.pallas_call(
        _edge_body,
        grid=(E // EBLK,),
        in_specs=[pl.BlockSpec((EBLK, DE), lambda i: (i, 0))]
                 + [_full_spec(w.shape) for w in ew],
        out_specs=[pl.BlockSpec((EBLK, HID), lambda i: (i, 0))] * 2,
        out_shape=[jax.ShapeDtypeStruct((E, HID), jnp.float32)] * 2,
    )(edge_attr, *ew)

    NB = 2000
    nw_specs = [_full_spec(w.shape) for w in
                (p["Win1"], p["bin1"], p["gin"], p["bbn_in"], p["Win2"],
                 p["bin2"], p["Wgat"], asrc_w, adst_w)]
    h, xh2d, a_src, a_dst, gmax_tab = pl.pallas_call(
        _node_body,
        grid=(N // NB,),
        in_specs=[pl.BlockSpec((NB, DF), lambda i: (i, 0))] + nw_specs,
        out_specs=[pl.BlockSpec((NB, HID), lambda i: (i, 0)),
                   pl.BlockSpec((NB, HEADS * HID), lambda i: (i, 0)),
                   pl.BlockSpec((NB, HEADS), lambda i: (i, 0)),
                   pl.BlockSpec((NB, HEADS), lambda i: (i, 0)),
                   pl.BlockSpec((1, 16), lambda i: (0, 0))],
        out_shape=[jax.ShapeDtypeStruct((N, HID), jnp.float32),
                   jax.ShapeDtypeStruct((N, HEADS * HID), jnp.float32),
                   jax.ShapeDtypeStruct((N, HEADS), jnp.float32),
                   jax.ShapeDtypeStruct((N, HEADS), jnp.float32),
                   jax.ShapeDtypeStruct((1, 16), jnp.float32)],
    )(x, p["Win1"], p["bin1"], p["gin"], p["bbn_in"], p["Win2"], p["bin2"],
      p["Wgat"], asrc_w, adst_w)

    ae_e, den_parts = _gat_p1a_sc(a_src.reshape(-1), a_dst.reshape(-1),
                                  gmax_tab, src, dst)
    dsum = den_parts.sum(axis=0).reshape(N, HEADS)

    ae_loop, invden = pl.pallas_call(
        _invden_body,
        out_shape=[jax.ShapeDtypeStruct((N, HEADS), jnp.float32),
                   jax.ShapeDtypeStruct((N, HEADS), jnp.float32)],
    )(a_src, a_dst, gmax_tab, dsum)

    att16 = _gat_p2a_sc(ae_e, invden.reshape(-1), dst)
    h1_parts = _gat_p2b_sc(xh2d, att16, src, dst)

    h1 = pl.pallas_call(
        _h1_body,
        grid=(N // NB,),
        in_specs=[pl.BlockSpec((NB, HID), lambda i: (i, 0)),
                  pl.BlockSpec((NB, HID), lambda i: (i, 0)),
                  pl.BlockSpec((NB, HEADS), lambda i: (i, 0)),
                  pl.BlockSpec((NB, HEADS), lambda i: (i, 0)),
                  pl.BlockSpec((NB, HEADS * HID), lambda i: (i, 0)),
                  _full_spec(p["bgat"].shape)],
        out_specs=pl.BlockSpec((NB, HID), lambda i: (i, 0)),
        out_shape=jax.ShapeDtypeStruct((N, HID), jnp.float32),
    )(h1_parts[0], h1_parts[1], ae_loop, invden, xh2d, p["bgat"])

    hw_specs = [_full_spec(w.shape) for w in
                (p["Wh1"], p["bh1"], p["gh"], p["bbn_h"], p["Wh2"], p["bh2"])]

    def gine(hn, el):
        parts = _gine_agg_sc(hn, el, src, dst)
        return pl.pallas_call(
            _hidden_body,
            grid=(N // NB,),
            in_specs=[pl.BlockSpec((NB, HID), lambda i: (i, 0))] * 3
                     + hw_specs,
            out_specs=pl.BlockSpec((NB, HID), lambda i: (i, 0)),
            out_shape=jax.ShapeDtypeStruct((N, HID), jnp.float32),
        )(hn, parts[0], parts[1], p["Wh1"], p["bh1"], p["gh"], p["bbn_h"],
          p["Wh2"], p["bh2"])

    h2 = gine(h1, el2)
    h3 = gine(h2, el3)

    out = pl.pallas_call(
        _pool_fc_body,
        out_shape=jax.ShapeDtypeStruct((G, OUT), jnp.float32),
    )(h1, h2, h3, batch.reshape(1, N), p["Wfc1"], p["bfc1"],
      p["Wfc2"], p["bfc2"])
    return out
